# Initial kernel scaffold; baseline (speedup 1.0000x reference)
import jax
import jax.numpy as jnp
from jax.experimental import pallas as pl


def kernel(features, edge_index, W0, b0, W1, b1, W2, b2):
    raise NotImplementedError("write your pallas kernel here")


# speedup vs baseline: 7.9805x; 7.9805x over previous
import functools

import jax
import jax.numpy as jnp
from jax import lax
from jax.experimental import pallas as pl
from jax.experimental.pallas import tpu as pltpu
from jax.experimental.pallas import tpu_sc as plsc

N_NODES = 10000
N_EDGES = 320000
D_FEAT = 128
D_HID = 128
N_CLS = 64
K_LAYERS = 8
ALPHA = 1.0
LAMBD = 1.0
COEF = 1.0 / (1.0 + ALPHA * LAMBD)
K1 = COEF * ALPHA * LAMBD
K2 = COEF

NC = 2
NS = 16
NW = NC * NS
NPAD = 10240
SLICE = NPAD // NS
CHUNK = 128
EPT = 10240
CPT = EPT // CHUNK
TOT_E = NW * EPT
NBUF = 4

_mesh = plsc.VectorSubcoreMesh(core_axis_name="c", subcore_axis_name="s",
                               num_cores=NC, num_subcores=NS)
_sc_params = pltpu.CompilerParams(use_tc_tiling_on_sc=False)


def _deg_body(src_hbm, dst_hbm, ones_hbm, zeros_hbm, dego_hbm, degi_hbm,
              srcv, dstv, onesv, dego_sh, degi_sh):
    c = lax.axis_index("c")
    s = lax.axis_index("s")
    w = c * NS + s
    pltpu.sync_copy(src_hbm.at[pl.ds(w * CPT, CPT)], srcv)
    pltpu.sync_copy(dst_hbm.at[pl.ds(w * CPT, CPT)], dstv)
    pltpu.sync_copy(ones_hbm, onesv)
    pltpu.sync_copy(zeros_hbm.at[pl.ds(s * SLICE, SLICE)],
                    dego_sh.at[pl.ds(s * SLICE, SLICE)])
    pltpu.sync_copy(zeros_hbm.at[pl.ds(s * SLICE, SLICE)],
                    degi_sh.at[pl.ds(s * SLICE, SLICE)])
    plsc.subcore_barrier()

    def chunk(j, carry):
        pltpu.sync_copy(onesv, dego_sh.at[srcv.at[j]], add=True)
        pltpu.sync_copy(onesv, degi_sh.at[dstv.at[j]], add=True)
        return carry

    lax.fori_loop(0, CPT, chunk, 0)
    plsc.subcore_barrier()
    pltpu.sync_copy(dego_sh.at[pl.ds(s * SLICE, SLICE)],
                    dego_hbm.at[c, pl.ds(s * SLICE, SLICE)])
    pltpu.sync_copy(degi_sh.at[pl.ds(s * SLICE, SLICE)],
                    degi_hbm.at[c, pl.ds(s * SLICE, SLICE)])


_deg_call = pl.kernel(
    _deg_body,
    out_type=(jax.ShapeDtypeStruct((NC, NPAD, 16), jnp.float32),
              jax.ShapeDtypeStruct((NC, NPAD, 16), jnp.float32)),
    mesh=_mesh,
    scratch_types=[
        pltpu.VMEM((CPT, CHUNK), jnp.int32),
        pltpu.VMEM((CPT, CHUNK), jnp.int32),
        pltpu.VMEM((CHUNK, 16), jnp.float32),
        pltpu.VMEM_SHARED((NPAD, 16), jnp.float32),
        pltpu.VMEM_SHARED((NPAD, 16), jnp.float32),
    ],
    compiler_params=_sc_params,
)


def _edge_body(u_hbm, src_hbm, dst_hbm, zeros_hbm, agg_hbm,
               srcv, dstv, rowsv, agg_sh, sem):
    c = lax.axis_index("c")
    s = lax.axis_index("s")
    w = c * NS + s
    pltpu.sync_copy(src_hbm.at[pl.ds(w * CPT, CPT)], srcv)
    pltpu.sync_copy(dst_hbm.at[pl.ds(w * CPT, CPT)], dstv)
    pltpu.sync_copy(zeros_hbm.at[pl.ds(s * SLICE, SLICE)],
                    agg_sh.at[pl.ds(s * SLICE, SLICE)])
    plsc.subcore_barrier()

    def outer(o, carry):
        descs = []
        for t in range(NBUF):
            descs.append(pltpu.async_copy(
                u_hbm.at[srcv.at[o * NBUF + t]], rowsv.at[t], sem))
        for d in descs:
            d.wait()
        for t in range(NBUF):
            pltpu.sync_copy(rowsv.at[t], agg_sh.at[dstv.at[o * NBUF + t]],
                            add=True)
        return carry

    lax.fori_loop(0, CPT // NBUF, outer, 0)
    plsc.subcore_barrier()
    pltpu.sync_copy(agg_sh.at[pl.ds(s * SLICE, SLICE)],
                    agg_hbm.at[c, pl.ds(s * SLICE, SLICE)])


_edge_call = pl.kernel(
    _edge_body,
    out_type=jax.ShapeDtypeStruct((NC, NPAD, N_CLS), jnp.float32),
    mesh=_mesh,
    scratch_types=[
        pltpu.VMEM((CPT, CHUNK), jnp.int32),
        pltpu.VMEM((CPT, CHUNK), jnp.int32),
        pltpu.VMEM((NBUF, CHUNK, N_CLS), jnp.float32),
        pltpu.VMEM_SHARED((NPAD, N_CLS), jnp.float32),
        pltpu.SemaphoreType.DMA,
    ],
    compiler_params=_sc_params,
)


_RB = 1000
_RBP = 1024


def _prep_body(x_ref, w0_ref, b0_ref, w1_ref, dgo_ref, dgi_ref,
               u0_ref, c_ref, ia_ref):
    h0 = jnp.dot(x_ref[...], w0_ref[...],
                 preferred_element_type=jnp.float32) + b0_ref[...]
    g0 = jnp.dot(h0, w1_ref[...], preferred_element_type=jnp.float32)
    dgo = jnp.maximum(dgo_ref[0][:, 0:1] + dgo_ref[1][:, 0:1], 1.0)
    dgi = jnp.maximum(dgi_ref[0][:, 0:1] + dgi_ref[1][:, 0:1], 1.0)
    a = lax.rsqrt(dgo)
    b = lax.rsqrt(dgi)
    c_ref[...] = a * b
    ia_ref[...] = jnp.sqrt(dgo)
    u0_ref[...] = a * g0


_prep_call = pl.pallas_call(
    _prep_body,
    grid=(N_NODES // _RB,),
    in_specs=[
        pl.BlockSpec((_RB, D_FEAT), lambda i: (i, 0)),
        pl.BlockSpec((D_FEAT, D_HID), lambda i: (0, 0)),
        pl.BlockSpec((1, D_HID), lambda i: (0, 0)),
        pl.BlockSpec((D_HID, N_CLS), lambda i: (0, 0)),
        pl.BlockSpec((NC, _RB, 16), lambda i: (0, i, 0)),
        pl.BlockSpec((NC, _RB, 16), lambda i: (0, i, 0)),
    ],
    out_specs=(
        pl.BlockSpec((_RB, N_CLS), lambda i: (i, 0)),
        pl.BlockSpec((_RB, 1), lambda i: (i, 0)),
        pl.BlockSpec((_RB, 1), lambda i: (i, 0)),
    ),
    out_shape=(
        jax.ShapeDtypeStruct((N_NODES, N_CLS), jnp.float32),
        jax.ShapeDtypeStruct((N_NODES, 1), jnp.float32),
        jax.ShapeDtypeStruct((N_NODES, 1), jnp.float32),
    ),
)


def _upd_body(agg_ref, c_ref, u0_ref, out_ref):
    ssum = agg_ref[0] + agg_ref[1]
    out_ref[...] = K1 * c_ref[...] * ssum + K2 * u0_ref[...]


_upd_call = pl.pallas_call(
    _upd_body,
    grid=(NPAD // _RBP,),
    in_specs=[
        pl.BlockSpec((NC, _RBP, N_CLS), lambda i: (0, i, 0)),
        pl.BlockSpec((_RBP, 1), lambda i: (i, 0)),
        pl.BlockSpec((_RBP, N_CLS), lambda i: (i, 0)),
    ],
    out_specs=pl.BlockSpec((_RBP, N_CLS), lambda i: (i, 0)),
    out_shape=jax.ShapeDtypeStruct((NPAD, N_CLS), jnp.float32),
)


def _fin_body(agg_ref, c_ref, u0_ref, ia_ref, b1_ref, w2_ref, b2_ref, o_ref):
    ssum = agg_ref[0] + agg_ref[1]
    u8 = K1 * c_ref[...] * ssum + K2 * u0_ref[...]
    g8 = ia_ref[...] * u8
    z = jnp.maximum(g8 + b1_ref[...], 0.0)
    o_ref[...] = jnp.dot(z, w2_ref[...],
                         preferred_element_type=jnp.float32) + b2_ref[...]


_fin_call = pl.pallas_call(
    _fin_body,
    grid=(N_NODES // _RB,),
    in_specs=[
        pl.BlockSpec((NC, _RB, N_CLS), lambda i: (0, i, 0)),
        pl.BlockSpec((_RB, 1), lambda i: (i, 0)),
        pl.BlockSpec((_RB, N_CLS), lambda i: (i, 0)),
        pl.BlockSpec((_RB, 1), lambda i: (i, 0)),
        pl.BlockSpec((1, N_CLS), lambda i: (0, 0)),
        pl.BlockSpec((N_CLS, N_CLS), lambda i: (0, 0)),
        pl.BlockSpec((1, N_CLS), lambda i: (0, 0)),
    ],
    out_specs=pl.BlockSpec((_RB, N_CLS), lambda i: (i, 0)),
    out_shape=jax.ShapeDtypeStruct((N_NODES, N_CLS), jnp.float32),
)


def kernel(features, edge_index, W0, b0, W1, b1, W2, b2):
    src = jnp.asarray(edge_index[0], jnp.int32)
    dst = jnp.asarray(edge_index[1], jnp.int32)
    pad = jnp.full((TOT_E - N_EDGES,), NPAD - 1, jnp.int32)
    src2 = jnp.concatenate([src, pad]).reshape(NW * CPT, CHUNK)
    dst2 = jnp.concatenate([dst, pad]).reshape(NW * CPT, CHUNK)

    ones16 = jnp.ones((CHUNK, 16), jnp.float32)
    zeros16 = jnp.zeros((NPAD, 16), jnp.float32)
    zeros64 = jnp.zeros((NPAD, N_CLS), jnp.float32)

    dego_p, degi_p = _deg_call(src2, dst2, ones16, zeros16)
    u0, cvec, inva = _prep_call(features, W0, b0.reshape(1, -1), W1,
                                dego_p, degi_p)
    u0p = jnp.pad(u0, ((0, NPAD - N_NODES), (0, 0)))
    cp = jnp.pad(cvec, ((0, NPAD - N_NODES), (0, 0)))

    u = u0p
    aggp = None
    for k in range(K_LAYERS):
        aggp = _edge_call(u, src2, dst2, zeros64)
        if k < K_LAYERS - 1:
            u = _upd_call(aggp, cp, u0p)

    return _fin_call(aggp, cvec, u0, inva, b1.reshape(1, -1), W2,
                     b2.reshape(1, -1))


# speedup vs baseline: 9.0361x; 1.1323x over previous
import functools

import jax
import jax.numpy as jnp
from jax import lax
from jax.experimental import pallas as pl
from jax.experimental.pallas import tpu as pltpu
from jax.experimental.pallas import tpu_sc as plsc

N_NODES = 10000
N_EDGES = 320000
D_FEAT = 128
D_HID = 128
N_CLS = 64
K_LAYERS = 8
ALPHA = 1.0
LAMBD = 1.0
COEF = 1.0 / (1.0 + ALPHA * LAMBD)
K1 = COEF * ALPHA * LAMBD
K2 = COEF

NC = 2
NS = 16
NW = NC * NS
NPAD = 10240
SLICE = NPAD // NS
CHUNK = 128
EPT = 10240
CPT = EPT // CHUNK
TOT_E = NW * EPT
NBUF = 4

_mesh = plsc.VectorSubcoreMesh(core_axis_name="c", subcore_axis_name="s",
                               num_cores=NC, num_subcores=NS)
_sc_params = pltpu.CompilerParams(use_tc_tiling_on_sc=False)


def _deg_body(src_hbm, dst_hbm, ones_hbm, zeros_hbm, dego_hbm, degi_hbm,
              srcv, dstv, onesv, dego_sh, degi_sh, dsem):
    c = lax.axis_index("c")
    s = lax.axis_index("s")
    w = c * NS + s
    pltpu.sync_copy(src_hbm.at[pl.ds(w * CPT, CPT)], srcv)
    pltpu.sync_copy(dst_hbm.at[pl.ds(w * CPT, CPT)], dstv)
    pltpu.sync_copy(ones_hbm, onesv)
    pltpu.sync_copy(zeros_hbm.at[pl.ds(s * SLICE, SLICE)],
                    dego_sh.at[pl.ds(s * SLICE, SLICE)])
    pltpu.sync_copy(zeros_hbm.at[pl.ds(s * SLICE, SLICE)],
                    degi_sh.at[pl.ds(s * SLICE, SLICE)])
    plsc.subcore_barrier()

    def chunk(j, carry):
        pltpu.async_copy(onesv, dego_sh.at[srcv.at[j]], dsem, add=True)
        pltpu.async_copy(onesv, degi_sh.at[dstv.at[j]], dsem, add=True)
        return carry

    lax.fori_loop(0, CPT, chunk, 0)

    def drain(j, carry):
        pltpu.make_async_copy(ones_hbm, onesv, dsem).wait()
        return carry

    lax.fori_loop(0, 2 * CPT, drain, 0)
    plsc.subcore_barrier()
    pltpu.sync_copy(dego_sh.at[pl.ds(s * SLICE, SLICE)],
                    dego_hbm.at[c, pl.ds(s * SLICE, SLICE)])
    pltpu.sync_copy(degi_sh.at[pl.ds(s * SLICE, SLICE)],
                    degi_hbm.at[c, pl.ds(s * SLICE, SLICE)])


_deg_call = pl.kernel(
    _deg_body,
    out_type=(jax.ShapeDtypeStruct((NC, NPAD, 16), jnp.float32),
              jax.ShapeDtypeStruct((NC, NPAD, 16), jnp.float32)),
    mesh=_mesh,
    scratch_types=[
        pltpu.VMEM((CPT, CHUNK), jnp.int32),
        pltpu.VMEM((CPT, CHUNK), jnp.int32),
        pltpu.VMEM((CHUNK, 16), jnp.float32),
        pltpu.VMEM_SHARED((NPAD, 16), jnp.float32),
        pltpu.VMEM_SHARED((NPAD, 16), jnp.float32),
        pltpu.SemaphoreType.DMA,
    ],
    compiler_params=_sc_params,
)


def _edge_body(u_hbm, src_hbm, dst_hbm, zeros_hbm, agg_hbm,
               srcv, dstv, rowsv, agg_sh, gsem_a, gsem_b, ssem_a, ssem_b):
    c = lax.axis_index("c")
    s = lax.axis_index("s")
    w = c * NS + s
    pltpu.sync_copy(src_hbm.at[pl.ds(w * CPT, CPT)], srcv)
    pltpu.sync_copy(dst_hbm.at[pl.ds(w * CPT, CPT)], dstv)
    pltpu.sync_copy(zeros_hbm.at[pl.ds(s * SLICE, SLICE)],
                    agg_sh.at[pl.ds(s * SLICE, SLICE)])
    plsc.subcore_barrier()

    def fire_g(g, bank, sem):
        for t in range(NBUF):
            pltpu.async_copy(u_hbm.at[srcv.at[g * NBUF + t]],
                             rowsv.at[bank, t], sem)

    def fire_s(g, bank, sem):
        for t in range(NBUF):
            pltpu.async_copy(rowsv.at[bank, t],
                             agg_sh.at[dstv.at[g * NBUF + t]], sem, add=True)

    def wait_n(sem):
        for _ in range(NBUF):
            pltpu.make_async_copy(u_hbm.at[pl.ds(0, CHUNK)],
                                  rowsv.at[0, 0], sem).wait()

    ngroups = CPT // NBUF
    fire_g(0, 0, gsem_a)
    fire_g(1, 1, gsem_b)
    wait_n(gsem_a)
    fire_s(0, 0, ssem_a)

    def body(ii, carry):
        g0 = ii * 2
        wait_n(ssem_a)
        fire_g(g0, 0, gsem_a)
        wait_n(gsem_b)
        fire_s(g0 - 1, 1, ssem_b)
        wait_n(ssem_b)
        fire_g(g0 + 1, 1, gsem_b)
        wait_n(gsem_a)
        fire_s(g0, 0, ssem_a)
        return carry

    lax.fori_loop(1, ngroups // 2, body, 0)
    wait_n(ssem_a)
    wait_n(gsem_b)
    fire_s(ngroups - 1, 1, ssem_b)
    wait_n(ssem_b)
    plsc.subcore_barrier()
    pltpu.sync_copy(agg_sh.at[pl.ds(s * SLICE, SLICE)],
                    agg_hbm.at[c, pl.ds(s * SLICE, SLICE)])


_edge_call = pl.kernel(
    _edge_body,
    out_type=jax.ShapeDtypeStruct((NC, NPAD, N_CLS), jnp.float32),
    mesh=_mesh,
    scratch_types=[
        pltpu.VMEM((CPT, CHUNK), jnp.int32),
        pltpu.VMEM((CPT, CHUNK), jnp.int32),
        pltpu.VMEM((2, NBUF, CHUNK, N_CLS), jnp.float32),
        pltpu.VMEM_SHARED((NPAD, N_CLS), jnp.float32),
        pltpu.SemaphoreType.DMA,
        pltpu.SemaphoreType.DMA,
        pltpu.SemaphoreType.DMA,
        pltpu.SemaphoreType.DMA,
    ],
    compiler_params=_sc_params,
)


_RB = 1000
_RBP = 1024


def _prep_body(x_ref, w0_ref, b0_ref, w1_ref, dgo_ref, dgi_ref,
               u0_ref, c_ref, ia_ref):
    h0 = jnp.dot(x_ref[...], w0_ref[...],
                 preferred_element_type=jnp.float32) + b0_ref[...]
    g0 = jnp.dot(h0, w1_ref[...], preferred_element_type=jnp.float32)
    dgo = jnp.maximum(dgo_ref[0][:, 0:1] + dgo_ref[1][:, 0:1], 1.0)
    dgi = jnp.maximum(dgi_ref[0][:, 0:1] + dgi_ref[1][:, 0:1], 1.0)
    a = lax.rsqrt(dgo)
    b = lax.rsqrt(dgi)
    c_ref[...] = a * b
    ia_ref[...] = jnp.sqrt(dgo)
    u0_ref[...] = a * g0


_prep_call = pl.pallas_call(
    _prep_body,
    grid=(N_NODES // _RB,),
    in_specs=[
        pl.BlockSpec((_RB, D_FEAT), lambda i: (i, 0)),
        pl.BlockSpec((D_FEAT, D_HID), lambda i: (0, 0)),
        pl.BlockSpec((1, D_HID), lambda i: (0, 0)),
        pl.BlockSpec((D_HID, N_CLS), lambda i: (0, 0)),
        pl.BlockSpec((NC, _RB, 16), lambda i: (0, i, 0)),
        pl.BlockSpec((NC, _RB, 16), lambda i: (0, i, 0)),
    ],
    out_specs=(
        pl.BlockSpec((_RB, N_CLS), lambda i: (i, 0)),
        pl.BlockSpec((_RB, 1), lambda i: (i, 0)),
        pl.BlockSpec((_RB, 1), lambda i: (i, 0)),
    ),
    out_shape=(
        jax.ShapeDtypeStruct((N_NODES, N_CLS), jnp.float32),
        jax.ShapeDtypeStruct((N_NODES, 1), jnp.float32),
        jax.ShapeDtypeStruct((N_NODES, 1), jnp.float32),
    ),
)


def _upd_body(agg_ref, c_ref, u0_ref, out_ref):
    ssum = agg_ref[0] + agg_ref[1]
    out_ref[...] = K1 * c_ref[...] * ssum + K2 * u0_ref[...]


_upd_call = pl.pallas_call(
    _upd_body,
    grid=(NPAD // _RBP,),
    in_specs=[
        pl.BlockSpec((NC, _RBP, N_CLS), lambda i: (0, i, 0)),
        pl.BlockSpec((_RBP, 1), lambda i: (i, 0)),
        pl.BlockSpec((_RBP, N_CLS), lambda i: (i, 0)),
    ],
    out_specs=pl.BlockSpec((_RBP, N_CLS), lambda i: (i, 0)),
    out_shape=jax.ShapeDtypeStruct((NPAD, N_CLS), jnp.float32),
)


def _fin_body(agg_ref, c_ref, u0_ref, ia_ref, b1_ref, w2_ref, b2_ref, o_ref):
    ssum = agg_ref[0] + agg_ref[1]
    u8 = K1 * c_ref[...] * ssum + K2 * u0_ref[...]
    g8 = ia_ref[...] * u8
    z = jnp.maximum(g8 + b1_ref[...], 0.0)
    o_ref[...] = jnp.dot(z, w2_ref[...],
                         preferred_element_type=jnp.float32) + b2_ref[...]


_fin_call = pl.pallas_call(
    _fin_body,
    grid=(N_NODES // _RB,),
    in_specs=[
        pl.BlockSpec((NC, _RB, N_CLS), lambda i: (0, i, 0)),
        pl.BlockSpec((_RB, 1), lambda i: (i, 0)),
        pl.BlockSpec((_RB, N_CLS), lambda i: (i, 0)),
        pl.BlockSpec((_RB, 1), lambda i: (i, 0)),
        pl.BlockSpec((1, N_CLS), lambda i: (0, 0)),
        pl.BlockSpec((N_CLS, N_CLS), lambda i: (0, 0)),
        pl.BlockSpec((1, N_CLS), lambda i: (0, 0)),
    ],
    out_specs=pl.BlockSpec((_RB, N_CLS), lambda i: (i, 0)),
    out_shape=jax.ShapeDtypeStruct((N_NODES, N_CLS), jnp.float32),
)


def kernel(features, edge_index, W0, b0, W1, b1, W2, b2):
    src = jnp.asarray(edge_index[0], jnp.int32)
    dst = jnp.asarray(edge_index[1], jnp.int32)
    pad = jnp.full((TOT_E - N_EDGES,), NPAD - 1, jnp.int32)
    src2 = jnp.concatenate([src, pad]).reshape(NW * CPT, CHUNK)
    dst2 = jnp.concatenate([dst, pad]).reshape(NW * CPT, CHUNK)

    ones16 = jnp.ones((CHUNK, 16), jnp.float32)
    zeros16 = jnp.zeros((NPAD, 16), jnp.float32)
    zeros64 = jnp.zeros((NPAD, N_CLS), jnp.float32)

    dego_p, degi_p = _deg_call(src2, dst2, ones16, zeros16)
    u0, cvec, inva = _prep_call(features, W0, b0.reshape(1, -1), W1,
                                dego_p, degi_p)
    u0p = jnp.pad(u0, ((0, NPAD - N_NODES), (0, 0)))
    cp = jnp.pad(cvec, ((0, NPAD - N_NODES), (0, 0)))

    u = u0p
    aggp = None
    for k in range(K_LAYERS):
        aggp = _edge_call(u, src2, dst2, zeros64)
        if k < K_LAYERS - 1:
            u = _upd_call(aggp, cp, u0p)

    return _fin_call(aggp, cvec, u0, inva, b1.reshape(1, -1), W2,
                     b2.reshape(1, -1))


# speedup vs baseline: 17.4531x; 1.9315x over previous
import functools

import jax
import jax.numpy as jnp
from jax import lax
from jax.experimental import pallas as pl
from jax.experimental.pallas import tpu as pltpu
from jax.experimental.pallas import tpu_sc as plsc

N_NODES = 10000
N_EDGES = 320000
D_FEAT = 128
D_HID = 128
N_CLS = 64
K_LAYERS = 8
ALPHA = 1.0
LAMBD = 1.0
COEF = 1.0 / (1.0 + ALPHA * LAMBD)
K1 = COEF * ALPHA * LAMBD
K2 = COEF

NC = 2
NS = 16
NW = NC * NS
HC = N_CLS // NC
NPAD = 10240
SLICE = NPAD // NS
CHUNK = 128
TOT_E = 327680
EPT = TOT_E // NS
CPT = EPT // CHUNK
DEG_CPT = TOT_E // NW // CHUNK
NBUF = 4

_mesh = plsc.VectorSubcoreMesh(core_axis_name="c", subcore_axis_name="s",
                               num_cores=NC, num_subcores=NS)
_sc_params = pltpu.CompilerParams(use_tc_tiling_on_sc=False)


def _deg_body(src_hbm, dst_hbm, ones_hbm, zeros_hbm, dego_hbm, degi_hbm,
              srcv, dstv, onesv, dego_sh, degi_sh, dsem):
    c = lax.axis_index("c")
    s = lax.axis_index("s")
    w = c * NS + s
    pltpu.sync_copy(src_hbm.at[pl.ds(w * DEG_CPT, DEG_CPT)], srcv)
    pltpu.sync_copy(dst_hbm.at[pl.ds(w * DEG_CPT, DEG_CPT)], dstv)
    pltpu.sync_copy(ones_hbm, onesv)
    pltpu.sync_copy(zeros_hbm.at[pl.ds(s * SLICE, SLICE)],
                    dego_sh.at[pl.ds(s * SLICE, SLICE)])
    pltpu.sync_copy(zeros_hbm.at[pl.ds(s * SLICE, SLICE)],
                    degi_sh.at[pl.ds(s * SLICE, SLICE)])
    plsc.subcore_barrier()

    def chunk(j, carry):
        pltpu.async_copy(onesv, dego_sh.at[srcv.at[j]], dsem, add=True)
        pltpu.async_copy(onesv, degi_sh.at[dstv.at[j]], dsem, add=True)
        return carry

    lax.fori_loop(0, DEG_CPT, chunk, 0)

    def drain(j, carry):
        pltpu.make_async_copy(ones_hbm, onesv, dsem).wait()
        return carry

    lax.fori_loop(0, 2 * DEG_CPT, drain, 0)
    plsc.subcore_barrier()
    pltpu.sync_copy(dego_sh.at[pl.ds(s * SLICE, SLICE)],
                    dego_hbm.at[c, pl.ds(s * SLICE, SLICE)])
    pltpu.sync_copy(degi_sh.at[pl.ds(s * SLICE, SLICE)],
                    degi_hbm.at[c, pl.ds(s * SLICE, SLICE)])


_deg_call = pl.kernel(
    _deg_body,
    out_type=(jax.ShapeDtypeStruct((NC, NPAD, 16), jnp.float32),
              jax.ShapeDtypeStruct((NC, NPAD, 16), jnp.float32)),
    mesh=_mesh,
    scratch_types=[
        pltpu.VMEM((DEG_CPT, CHUNK), jnp.int32),
        pltpu.VMEM((DEG_CPT, CHUNK), jnp.int32),
        pltpu.VMEM((CHUNK, 16), jnp.float32),
        pltpu.VMEM_SHARED((NPAD, 16), jnp.float32),
        pltpu.VMEM_SHARED((NPAD, 16), jnp.float32),
        pltpu.SemaphoreType.DMA,
    ],
    compiler_params=_sc_params,
)


def _edge_body(u_hbm, src_hbm, dst_hbm, zeros_hbm, agg_hbm,
               srcv, dstv, rowsv, u_sh, agg_sh,
               gsem_a, gsem_b, ssem_a, ssem_b):
    c = lax.axis_index("c")
    s = lax.axis_index("s")
    pltpu.sync_copy(src_hbm.at[pl.ds(s * CPT, CPT)], srcv)
    pltpu.sync_copy(dst_hbm.at[pl.ds(s * CPT, CPT)], dstv)
    pltpu.sync_copy(u_hbm.at[c, pl.ds(s * SLICE, SLICE)],
                    u_sh.at[pl.ds(s * SLICE, SLICE)])
    pltpu.sync_copy(zeros_hbm.at[pl.ds(s * SLICE, SLICE)],
                    agg_sh.at[pl.ds(s * SLICE, SLICE)])
    plsc.subcore_barrier()

    def fire_g(g, bank, sem):
        for t in range(NBUF):
            pltpu.async_copy(u_sh.at[srcv.at[g * NBUF + t]],
                             rowsv.at[bank, t], sem)

    def fire_s(g, bank, sem):
        for t in range(NBUF):
            pltpu.async_copy(rowsv.at[bank, t],
                             agg_sh.at[dstv.at[g * NBUF + t]], sem, add=True)

    def wait_n(sem):
        for _ in range(NBUF):
            pltpu.make_async_copy(u_hbm.at[0, pl.ds(0, CHUNK)],
                                  rowsv.at[0, 0], sem).wait()

    ngroups = CPT // NBUF
    fire_g(0, 0, gsem_a)
    fire_g(1, 1, gsem_b)
    wait_n(gsem_a)
    fire_s(0, 0, ssem_a)

    def body(ii, carry):
        g0 = ii * 2
        wait_n(ssem_a)
        fire_g(g0, 0, gsem_a)
        wait_n(gsem_b)
        fire_s(g0 - 1, 1, ssem_b)
        wait_n(ssem_b)
        fire_g(g0 + 1, 1, gsem_b)
        wait_n(gsem_a)
        fire_s(g0, 0, ssem_a)
        return carry

    lax.fori_loop(1, ngroups // 2, body, 0)
    wait_n(ssem_a)
    wait_n(gsem_b)
    fire_s(ngroups - 1, 1, ssem_b)
    wait_n(ssem_b)
    plsc.subcore_barrier()
    pltpu.sync_copy(agg_sh.at[pl.ds(s * SLICE, SLICE)],
                    agg_hbm.at[c, pl.ds(s * SLICE, SLICE)])


_edge_call = pl.kernel(
    _edge_body,
    out_type=jax.ShapeDtypeStruct((NC, NPAD, HC), jnp.float32),
    mesh=_mesh,
    scratch_types=[
        pltpu.VMEM((CPT, CHUNK), jnp.int32),
        pltpu.VMEM((CPT, CHUNK), jnp.int32),
        pltpu.VMEM((2, NBUF, CHUNK, HC), jnp.float32),
        pltpu.VMEM_SHARED((NPAD, HC), jnp.float32),
        pltpu.VMEM_SHARED((NPAD, HC), jnp.float32),
        pltpu.SemaphoreType.DMA,
        pltpu.SemaphoreType.DMA,
        pltpu.SemaphoreType.DMA,
        pltpu.SemaphoreType.DMA,
    ],
    compiler_params=_sc_params,
)


_RB = 1000
_RBP = 1024


def _prep_body(x_ref, w0_ref, b0_ref, w1_ref, dgo_ref, dgi_ref,
               u0_ref, c_ref, ia_ref):
    h0 = jnp.dot(x_ref[...], w0_ref[...],
                 preferred_element_type=jnp.float32) + b0_ref[...]
    g0 = jnp.dot(h0, w1_ref[...], preferred_element_type=jnp.float32)
    dgo = jnp.maximum(dgo_ref[0][:, 0:1] + dgo_ref[1][:, 0:1], 1.0)
    dgi = jnp.maximum(dgi_ref[0][:, 0:1] + dgi_ref[1][:, 0:1], 1.0)
    a = lax.rsqrt(dgo)
    b = lax.rsqrt(dgi)
    c_ref[...] = a * b
    ia_ref[...] = jnp.sqrt(dgo)
    u0 = a * g0
    u0_ref[0] = u0[:, :HC]
    u0_ref[1] = u0[:, HC:]


_prep_call = pl.pallas_call(
    _prep_body,
    grid=(N_NODES // _RB,),
    in_specs=[
        pl.BlockSpec((_RB, D_FEAT), lambda i: (i, 0)),
        pl.BlockSpec((D_FEAT, D_HID), lambda i: (0, 0)),
        pl.BlockSpec((1, D_HID), lambda i: (0, 0)),
        pl.BlockSpec((D_HID, N_CLS), lambda i: (0, 0)),
        pl.BlockSpec((NC, _RB, 16), lambda i: (0, i, 0)),
        pl.BlockSpec((NC, _RB, 16), lambda i: (0, i, 0)),
    ],
    out_specs=(
        pl.BlockSpec((NC, _RB, HC), lambda i: (0, i, 0)),
        pl.BlockSpec((_RB, 1), lambda i: (i, 0)),
        pl.BlockSpec((_RB, 1), lambda i: (i, 0)),
    ),
    out_shape=(
        jax.ShapeDtypeStruct((NC, N_NODES, HC), jnp.float32),
        jax.ShapeDtypeStruct((N_NODES, 1), jnp.float32),
        jax.ShapeDtypeStruct((N_NODES, 1), jnp.float32),
    ),
)


def _upd_body(agg_ref, c_ref, u0_ref, out_ref):
    out_ref[...] = K1 * c_ref[...] * agg_ref[...] + K2 * u0_ref[...]


_upd_call = pl.pallas_call(
    _upd_body,
    grid=(NPAD // _RBP,),
    in_specs=[
        pl.BlockSpec((NC, _RBP, HC), lambda i: (0, i, 0)),
        pl.BlockSpec((_RBP, 1), lambda i: (i, 0)),
        pl.BlockSpec((NC, _RBP, HC), lambda i: (0, i, 0)),
    ],
    out_specs=pl.BlockSpec((NC, _RBP, HC), lambda i: (0, i, 0)),
    out_shape=jax.ShapeDtypeStruct((NC, NPAD, HC), jnp.float32),
)


def _fin_body(agg_ref, c_ref, u0_ref, ia_ref, b1_ref, w2_ref, b2_ref, o_ref):
    u8 = K1 * c_ref[...] * agg_ref[...] + K2 * u0_ref[...]
    g8 = ia_ref[...] * u8
    z = jnp.concatenate([g8[0], g8[1]], axis=1) + b1_ref[...]
    z = jnp.maximum(z, 0.0)
    o_ref[...] = jnp.dot(z, w2_ref[...],
                         preferred_element_type=jnp.float32) + b2_ref[...]


_fin_call = pl.pallas_call(
    _fin_body,
    grid=(N_NODES // _RB,),
    in_specs=[
        pl.BlockSpec((NC, _RB, HC), lambda i: (0, i, 0)),
        pl.BlockSpec((_RB, 1), lambda i: (i, 0)),
        pl.BlockSpec((NC, _RB, HC), lambda i: (0, i, 0)),
        pl.BlockSpec((_RB, 1), lambda i: (i, 0)),
        pl.BlockSpec((1, N_CLS), lambda i: (0, 0)),
        pl.BlockSpec((N_CLS, N_CLS), lambda i: (0, 0)),
        pl.BlockSpec((1, N_CLS), lambda i: (0, 0)),
    ],
    out_specs=pl.BlockSpec((_RB, N_CLS), lambda i: (i, 0)),
    out_shape=jax.ShapeDtypeStruct((N_NODES, N_CLS), jnp.float32),
)


def kernel(features, edge_index, W0, b0, W1, b1, W2, b2):
    src = jnp.asarray(edge_index[0], jnp.int32)
    dst = jnp.asarray(edge_index[1], jnp.int32)
    pad = jnp.full((TOT_E - N_EDGES,), NPAD - 1, jnp.int32)
    src2 = jnp.concatenate([src, pad]).reshape(TOT_E // CHUNK, CHUNK)
    dst2 = jnp.concatenate([dst, pad]).reshape(TOT_E // CHUNK, CHUNK)

    ones16 = jnp.ones((CHUNK, 16), jnp.float32)
    zeros16 = jnp.zeros((NPAD, 16), jnp.float32)
    zerosHC = jnp.zeros((NPAD, HC), jnp.float32)

    dego_p, degi_p = _deg_call(src2, dst2, ones16, zeros16)
    u0, cvec, inva = _prep_call(features, W0, b0.reshape(1, -1), W1,
                                dego_p, degi_p)
    u0p = jnp.pad(u0, ((0, 0), (0, NPAD - N_NODES), (0, 0)))
    cp = jnp.pad(cvec, ((0, NPAD - N_NODES), (0, 0)))

    u = u0p
    aggp = None
    for k in range(K_LAYERS):
        aggp = _edge_call(u, src2, dst2, zerosHC)
        if k < K_LAYERS - 1:
            u = _upd_call(aggp, cp, u0p)

    return _fin_call(aggp, cvec, u0, inva, b1.reshape(1, -1), W2,
                     b2.reshape(1, -1))


# speedup vs baseline: 23.9907x; 1.3746x over previous
import functools

import jax
import jax.numpy as jnp
from jax import lax
from jax.experimental import pallas as pl
from jax.experimental.pallas import tpu as pltpu
from jax.experimental.pallas import tpu_sc as plsc

N_NODES = 10000
N_EDGES = 320000
D_FEAT = 128
D_HID = 128
N_CLS = 64
K_LAYERS = 8
ALPHA = 1.0
LAMBD = 1.0
COEF = 1.0 / (1.0 + ALPHA * LAMBD)
K1 = COEF * ALPHA * LAMBD
K2 = COEF

NC = 2
NS = 16
NW = NC * NS
HC = N_CLS // NC
NPAD = 10240
NPAD_EXTRA = NPAD - N_NODES
SLICE = NPAD // NS
CHUNK = 128
TOT_E = 327680
EPT = TOT_E // NS
CPT = EPT // CHUNK
DEG_CPT = TOT_E // NW // CHUNK
NBUF = 2

_mesh = plsc.VectorSubcoreMesh(core_axis_name="c", subcore_axis_name="s",
                               num_cores=NC, num_subcores=NS)
_sc_params = pltpu.CompilerParams(use_tc_tiling_on_sc=False)


def _deg_body(src_hbm, dst_hbm, ones_hbm, zeros_hbm, deg_hbm,
              srcv, dstv, oneslo, oneshi, deg_sh, dsem):
    c = lax.axis_index("c")
    s = lax.axis_index("s")
    w = c * NS + s
    pltpu.sync_copy(src_hbm.at[pl.ds(w * DEG_CPT, DEG_CPT)], srcv)
    pltpu.sync_copy(dst_hbm.at[pl.ds(w * DEG_CPT, DEG_CPT)], dstv)
    pltpu.sync_copy(ones_hbm.at[0], oneslo)
    pltpu.sync_copy(ones_hbm.at[1], oneshi)
    pltpu.sync_copy(zeros_hbm.at[pl.ds(s * SLICE, SLICE)],
                    deg_sh.at[pl.ds(s * SLICE, SLICE)])
    plsc.subcore_barrier()

    def chunk(j, carry):
        pltpu.async_copy(oneslo, deg_sh.at[srcv.at[j]], dsem, add=True)
        pltpu.async_copy(oneshi, deg_sh.at[dstv.at[j]], dsem, add=True)
        return carry

    lax.fori_loop(0, DEG_CPT, chunk, 0)

    def drain(j, carry):
        pltpu.make_async_copy(ones_hbm.at[0], oneslo, dsem).wait()
        return carry

    lax.fori_loop(0, 2 * DEG_CPT, drain, 0)
    plsc.subcore_barrier()
    pltpu.sync_copy(deg_sh.at[pl.ds(s * SLICE, SLICE)],
                    deg_hbm.at[c, pl.ds(s * SLICE, SLICE)])


_deg_call = pl.kernel(
    _deg_body,
    out_type=jax.ShapeDtypeStruct((NC, NPAD, 16), jnp.float32),
    mesh=_mesh,
    scratch_types=[
        pltpu.VMEM((DEG_CPT, CHUNK), jnp.int32),
        pltpu.VMEM((DEG_CPT, CHUNK), jnp.int32),
        pltpu.VMEM((CHUNK, 16), jnp.float32),
        pltpu.VMEM((CHUNK, 16), jnp.float32),
        pltpu.VMEM_SHARED((NPAD, 16), jnp.float32),
        pltpu.SemaphoreType.DMA,
    ],
    compiler_params=_sc_params,
)


def _prop_body(u0_hbm, c_hbm, pk_hbm, zeros_hbm, u8_hbm,
               pkv, sring, dring, rowsv, u0v, aggv, cv, zerov, u_sh, agg_sh,
               gsem_a, gsem_b, ssem_a, ssem_b):
    c = lax.axis_index("c")
    s = lax.axis_index("s")
    nsl = pl.ds(s * SLICE, SLICE)
    pltpu.sync_copy(pk_hbm.at[pl.ds(s * CPT, CPT)], pkv)
    pltpu.sync_copy(u0_hbm.at[c, nsl], u0v)
    pltpu.sync_copy(u0_hbm.at[c, nsl], u_sh.at[nsl])
    pltpu.sync_copy(c_hbm.at[nsl], cv)
    pltpu.sync_copy(zeros_hbm, zerov)

    def zero_agg():
        for z in range(SLICE // CHUNK):
            pltpu.sync_copy(zerov,
                            agg_sh.at[pl.ds(s * SLICE + z * CHUNK, CHUNK)])

    zero_agg()
    plsc.subcore_barrier()

    def fire_g(g, bank, sem):
        for t in range(NBUF):
            j = g * NBUF + t
            for v in range(CHUNK // 16):
                slc = pl.ds(v * 16, 16)
                pk = pkv[j, slc]
                sring[bank, t, slc] = lax.shift_right_logical(pk, 14)
                dring[bank, t, slc] = lax.bitwise_and(pk, 16383)
            pltpu.async_copy(u_sh.at[sring.at[bank, t]],
                             rowsv.at[bank, t], sem)

    def fire_s(g, bank, sem):
        for t in range(NBUF):
            pltpu.async_copy(rowsv.at[bank, t],
                             agg_sh.at[dring.at[bank, t]], sem, add=True)

    def wait_n(sem):
        for _ in range(NBUF):
            pltpu.make_async_copy(u0_hbm.at[0, pl.ds(0, CHUNK)],
                                  rowsv.at[0, 0], sem).wait()

    ngroups = CPT // NBUF

    def one_iter(k, carry):
        fire_g(0, 0, gsem_a)
        fire_g(1, 1, gsem_b)
        wait_n(gsem_a)
        fire_s(0, 0, ssem_a)

        def body(ii, cc):
            g0 = ii * 2
            wait_n(ssem_a)
            fire_g(g0, 0, gsem_a)
            wait_n(gsem_b)
            fire_s(g0 - 1, 1, ssem_b)
            wait_n(ssem_b)
            fire_g(g0 + 1, 1, gsem_b)
            wait_n(gsem_a)
            fire_s(g0, 0, ssem_a)
            return cc

        lax.fori_loop(1, ngroups // 2, body, 0)
        wait_n(ssem_a)
        wait_n(gsem_b)
        fire_s(ngroups - 1, 1, ssem_b)
        wait_n(ssem_b)
        plsc.subcore_barrier()

        pltpu.sync_copy(agg_sh.at[nsl], aggv)
        zero_agg()

        def upd(n16, cc):
            cvec = cv[pl.ds(n16 * 16, 16)] * K1
            for j in range(16):
                n = n16 * 16 + j
                cn = cvec[j]
                for h in range(HC // 16):
                    slc = pl.ds(h * 16, 16)
                    aggv[n, slc] = cn * aggv[n, slc] + K2 * u0v[n, slc]
            return cc

        lax.fori_loop(0, SLICE // 16, upd, 0)
        pltpu.sync_copy(aggv, u_sh.at[nsl])
        plsc.subcore_barrier()
        return carry

    lax.fori_loop(0, K_LAYERS, one_iter, 0)
    pltpu.sync_copy(u_sh.at[nsl], u8_hbm.at[c, nsl])


_prop_call = pl.kernel(
    _prop_body,
    out_type=jax.ShapeDtypeStruct((NC, NPAD, HC), jnp.float32),
    mesh=_mesh,
    scratch_types=[
        pltpu.VMEM((CPT, CHUNK), jnp.int32),
        pltpu.VMEM((2, NBUF, CHUNK), jnp.int32),
        pltpu.VMEM((2, NBUF, CHUNK), jnp.int32),
        pltpu.VMEM((2, NBUF, CHUNK, HC), jnp.float32),
        pltpu.VMEM((SLICE, HC), jnp.float32),
        pltpu.VMEM((SLICE, HC), jnp.float32),
        pltpu.VMEM((SLICE,), jnp.float32),
        pltpu.VMEM((CHUNK, HC), jnp.float32),
        pltpu.VMEM_SHARED((NPAD, HC), jnp.float32),
        pltpu.VMEM_SHARED((NPAD, HC), jnp.float32),
        pltpu.SemaphoreType.DMA,
        pltpu.SemaphoreType.DMA,
        pltpu.SemaphoreType.DMA,
        pltpu.SemaphoreType.DMA,
    ],
    compiler_params=_sc_params,
)


_RB = 1000


def _prep_body(x_ref, w0_ref, b0_ref, w1_ref, deg_ref,
               u0_ref, c_ref, ia_ref):
    h0 = jnp.dot(x_ref[...], w0_ref[...],
                 preferred_element_type=jnp.float32) + b0_ref[...]
    g0 = jnp.dot(h0, w1_ref[...], preferred_element_type=jnp.float32)
    dgo = jnp.maximum(deg_ref[0][:, 0:1] + deg_ref[1][:, 0:1], 1.0)
    dgi = jnp.maximum(deg_ref[0][:, 8:9] + deg_ref[1][:, 8:9], 1.0)
    a = lax.rsqrt(dgo)
    b = lax.rsqrt(dgi)
    c_ref[...] = a * b
    ia_ref[...] = jnp.sqrt(dgo)
    u0 = a * g0
    u0_ref[0] = u0[:, :HC]
    u0_ref[1] = u0[:, HC:]


_prep_call = pl.pallas_call(
    _prep_body,
    grid=(N_NODES // _RB,),
    in_specs=[
        pl.BlockSpec((_RB, D_FEAT), lambda i: (i, 0)),
        pl.BlockSpec((D_FEAT, D_HID), lambda i: (0, 0)),
        pl.BlockSpec((1, D_HID), lambda i: (0, 0)),
        pl.BlockSpec((D_HID, N_CLS), lambda i: (0, 0)),
        pl.BlockSpec((NC, _RB, 16), lambda i: (0, i, 0)),
    ],
    out_specs=(
        pl.BlockSpec((NC, _RB, HC), lambda i: (0, i, 0)),
        pl.BlockSpec((_RB, 1), lambda i: (i, 0)),
        pl.BlockSpec((_RB, 1), lambda i: (i, 0)),
    ),
    out_shape=(
        jax.ShapeDtypeStruct((NC, N_NODES, HC), jnp.float32),
        jax.ShapeDtypeStruct((N_NODES, 1), jnp.float32),
        jax.ShapeDtypeStruct((N_NODES, 1), jnp.float32),
    ),
)


def _fin_body(u8_ref, ia_ref, b1_ref, w2_ref, b2_ref, o_ref):
    g8 = ia_ref[...] * jnp.concatenate([u8_ref[0], u8_ref[1]], axis=1)
    z = jnp.maximum(g8 + b1_ref[...], 0.0)
    o_ref[...] = jnp.dot(z, w2_ref[...],
                         preferred_element_type=jnp.float32) + b2_ref[...]


_fin_call = pl.pallas_call(
    _fin_body,
    grid=(N_NODES // _RB,),
    in_specs=[
        pl.BlockSpec((NC, _RB, HC), lambda i: (0, i, 0)),
        pl.BlockSpec((_RB, 1), lambda i: (i, 0)),
        pl.BlockSpec((1, N_CLS), lambda i: (0, 0)),
        pl.BlockSpec((N_CLS, N_CLS), lambda i: (0, 0)),
        pl.BlockSpec((1, N_CLS), lambda i: (0, 0)),
    ],
    out_specs=pl.BlockSpec((_RB, N_CLS), lambda i: (i, 0)),
    out_shape=jax.ShapeDtypeStruct((N_NODES, N_CLS), jnp.float32),
)


def kernel(features, edge_index, W0, b0, W1, b1, W2, b2):
    src = jnp.asarray(edge_index[0], jnp.int32)
    dst = jnp.asarray(edge_index[1], jnp.int32)
    npd = TOT_E - N_EDGES
    pad = N_NODES + (jnp.arange(npd, dtype=jnp.int32) % NPAD_EXTRA)
    src2 = jnp.concatenate([src, pad]).reshape(TOT_E // CHUNK, CHUNK)
    dst2 = jnp.concatenate([dst, pad]).reshape(TOT_E // CHUNK, CHUNK)

    colid = jnp.arange(16, dtype=jnp.int32)
    ones2 = jnp.stack([jnp.tile((colid < 8).astype(jnp.float32), (CHUNK, 1)),
                       jnp.tile((colid >= 8).astype(jnp.float32), (CHUNK, 1))])
    zeros16 = jnp.zeros((NPAD, 16), jnp.float32)
    zerosHC = jnp.zeros((CHUNK, HC), jnp.float32)

    deg_p = _deg_call(src2, dst2, ones2, zeros16)
    u0, cvec, inva = _prep_call(features, W0, b0.reshape(1, -1), W1, deg_p)
    u0p = jnp.pad(u0, ((0, 0), (0, NPAD_EXTRA), (0, 0)))
    cp = jnp.pad(cvec[:, 0], ((0, NPAD_EXTRA),))

    pk2 = src2 * 16384 + dst2
    u8 = _prop_call(u0p, cp, pk2, zerosHC)
    return _fin_call(u8, inva, b1.reshape(1, -1), W2, b2.reshape(1, -1))


# speedup vs baseline: 24.6396x; 1.0270x over previous
import functools

import jax
import jax.numpy as jnp
from jax import lax
from jax.experimental import pallas as pl
from jax.experimental.pallas import tpu as pltpu
from jax.experimental.pallas import tpu_sc as plsc

N_NODES = 10000
N_EDGES = 320000
D_FEAT = 128
D_HID = 128
N_CLS = 64
K_LAYERS = 8
ALPHA = 1.0
LAMBD = 1.0
COEF = 1.0 / (1.0 + ALPHA * LAMBD)
K1 = COEF * ALPHA * LAMBD
K2 = COEF

NC = 2
NS = 16
NW = NC * NS
HC = N_CLS // NC
NPAD = 10240
NPAD_EXTRA = NPAD - N_NODES
SLICE = NPAD // NS
CHUNK = 128
TOT_E = 327680
EPT = TOT_E // NS
CPT = EPT // CHUNK
DEG_CPT = TOT_E // NW // CHUNK
NBUF = 4

_mesh = plsc.VectorSubcoreMesh(core_axis_name="c", subcore_axis_name="s",
                               num_cores=NC, num_subcores=NS)
_sc_params = pltpu.CompilerParams(use_tc_tiling_on_sc=False)


def _deg_body(src_hbm, dst_hbm, ones_hbm, zeros_hbm, deg_hbm,
              srcv, dstv, oneslo, oneshi, deg_sh, dsem):
    c = lax.axis_index("c")
    s = lax.axis_index("s")
    w = c * NS + s
    pltpu.sync_copy(src_hbm.at[pl.ds(w * DEG_CPT, DEG_CPT)], srcv)
    pltpu.sync_copy(dst_hbm.at[pl.ds(w * DEG_CPT, DEG_CPT)], dstv)
    pltpu.sync_copy(ones_hbm.at[0], oneslo)
    pltpu.sync_copy(ones_hbm.at[1], oneshi)
    pltpu.sync_copy(zeros_hbm.at[pl.ds(s * SLICE, SLICE)],
                    deg_sh.at[pl.ds(s * SLICE, SLICE)])
    plsc.subcore_barrier()

    def chunk(j, carry):
        pltpu.async_copy(oneslo, deg_sh.at[srcv.at[j]], dsem, add=True)
        pltpu.async_copy(oneshi, deg_sh.at[dstv.at[j]], dsem, add=True)
        return carry

    lax.fori_loop(0, DEG_CPT, chunk, 0)

    def drain(j, carry):
        pltpu.make_async_copy(ones_hbm.at[0], oneslo, dsem).wait()
        return carry

    lax.fori_loop(0, 2 * DEG_CPT, drain, 0)
    plsc.subcore_barrier()
    pltpu.sync_copy(deg_sh.at[pl.ds(s * SLICE, SLICE)],
                    deg_hbm.at[c, pl.ds(s * SLICE, SLICE)])


_deg_call = pl.kernel(
    _deg_body,
    out_type=jax.ShapeDtypeStruct((NC, NPAD, 16), jnp.float32),
    mesh=_mesh,
    scratch_types=[
        pltpu.VMEM((DEG_CPT, CHUNK), jnp.int32),
        pltpu.VMEM((DEG_CPT, CHUNK), jnp.int32),
        pltpu.VMEM((CHUNK, 16), jnp.float32),
        pltpu.VMEM((CHUNK, 16), jnp.float32),
        pltpu.VMEM_SHARED((NPAD, 16), jnp.float32),
        pltpu.SemaphoreType.DMA,
    ],
    compiler_params=_sc_params,
)


def _prop_body(u0_hbm, c_hbm, pk_hbm, zeros_hbm, u8_hbm,
               pkv, sring, dring, rowsv, u0v, cv, zerov, u_sh, agg_sh,
               gsem_a, gsem_b, ssem_a, ssem_b):
    c = lax.axis_index("c")
    s = lax.axis_index("s")
    nsl = pl.ds(s * SLICE, SLICE)
    pltpu.sync_copy(pk_hbm.at[pl.ds(s * CPT, CPT)], pkv)
    pltpu.sync_copy(u0_hbm.at[c, nsl], u0v)
    pltpu.sync_copy(u0_hbm.at[c, nsl], u_sh.at[nsl])
    pltpu.sync_copy(c_hbm.at[nsl], cv)
    pltpu.sync_copy(zeros_hbm, zerov)

    def zero_agg():
        for z in range(SLICE // CHUNK):
            pltpu.sync_copy(zerov,
                            agg_sh.at[pl.ds(s * SLICE + z * CHUNK, CHUNK)])

    zero_agg()
    plsc.subcore_barrier()

    def fire_g(g, bank, sem):
        for t in range(NBUF):
            j = g * NBUF + t
            for v in range(CHUNK // 16):
                slc = pl.ds(v * 16, 16)
                pk = pkv[j, slc]
                sring[bank, t, slc] = lax.shift_right_logical(pk, 14)
                dring[bank, t, slc] = lax.bitwise_and(pk, 16383)
            pltpu.async_copy(u_sh.at[sring.at[bank, t]],
                             rowsv.at[bank, t], sem)

    def fire_s(g, bank, sem):
        for t in range(NBUF):
            pltpu.async_copy(rowsv.at[bank, t],
                             agg_sh.at[dring.at[bank, t]], sem, add=True)

    def wait_n(sem):
        for _ in range(NBUF):
            pltpu.make_async_copy(u0_hbm.at[0, pl.ds(0, CHUNK)],
                                  rowsv.at[0, 0], sem).wait()

    ngroups = CPT // NBUF

    def one_iter(k, carry):
        fire_g(0, 0, gsem_a)
        fire_g(1, 1, gsem_b)
        wait_n(gsem_a)
        fire_s(0, 0, ssem_a)

        def body(ii, cc):
            g0 = ii * 2
            wait_n(ssem_a)
            fire_g(g0, 0, gsem_a)
            wait_n(gsem_b)
            fire_s(g0 - 1, 1, ssem_b)
            wait_n(ssem_b)
            fire_g(g0 + 1, 1, gsem_b)
            wait_n(gsem_a)
            fire_s(g0, 0, ssem_a)
            return cc

        lax.fori_loop(1, ngroups // 2, body, 0)
        wait_n(ssem_a)
        wait_n(gsem_b)
        fire_s(ngroups - 1, 1, ssem_b)
        wait_n(ssem_b)
        plsc.subcore_barrier()

        pieces = [(p // NBUF, p % NBUF) for p in range(SLICE // CHUNK)]
        for p, (pb, pt) in enumerate(pieces):
            pltpu.sync_copy(agg_sh.at[pl.ds(s * SLICE + p * CHUNK, CHUNK)],
                            rowsv.at[pb, pt])
        zero_agg()

        for p, (pb, pt) in enumerate(pieces):
            def upd(n16, cc, p=p, pb=pb, pt=pt):
                cvec = cv[pl.ds(p * CHUNK + n16 * 16, 16)] * K1
                for j in range(16):
                    n = n16 * 16 + j
                    cn = cvec[j]
                    for h in range(HC // 16):
                        slc = pl.ds(h * 16, 16)
                        rowsv[pb, pt, n, slc] = (cn * rowsv[pb, pt, n, slc]
                                                 + K2 * u0v[p * CHUNK + n, slc])
                return cc

            lax.fori_loop(0, CHUNK // 16, upd, 0)
        for p, (pb, pt) in enumerate(pieces):
            pltpu.sync_copy(rowsv.at[pb, pt],
                            u_sh.at[pl.ds(s * SLICE + p * CHUNK, CHUNK)])
        plsc.subcore_barrier()
        return carry

    lax.fori_loop(0, K_LAYERS, one_iter, 0)
    pltpu.sync_copy(u_sh.at[nsl], u8_hbm.at[c, nsl])


_prop_call = pl.kernel(
    _prop_body,
    out_type=jax.ShapeDtypeStruct((NC, NPAD, HC), jnp.float32),
    mesh=_mesh,
    scratch_types=[
        pltpu.VMEM((CPT, CHUNK), jnp.int32),
        pltpu.VMEM((2, NBUF, CHUNK), jnp.int32),
        pltpu.VMEM((2, NBUF, CHUNK), jnp.int32),
        pltpu.VMEM((2, NBUF, CHUNK, HC), jnp.float32),
        pltpu.VMEM((SLICE, HC), jnp.float32),
        pltpu.VMEM((SLICE,), jnp.float32),
        pltpu.VMEM((CHUNK, HC), jnp.float32),
        pltpu.VMEM_SHARED((NPAD, HC), jnp.float32),
        pltpu.VMEM_SHARED((NPAD, HC), jnp.float32),
        pltpu.SemaphoreType.DMA,
        pltpu.SemaphoreType.DMA,
        pltpu.SemaphoreType.DMA,
        pltpu.SemaphoreType.DMA,
    ],
    compiler_params=_sc_params,
)


_RB = 1000
_RBP = 1024


def _prep_body(x_ref, w0_ref, b0_ref, w1_ref, deg_ref,
               u0_ref, c_ref, ia_ref):
    i = pl.program_id(0)
    h0 = jnp.dot(x_ref[...], w0_ref[...],
                 preferred_element_type=jnp.float32) + b0_ref[...]
    g0 = jnp.dot(h0, w1_ref[...], preferred_element_type=jnp.float32)
    dgo = jnp.maximum(deg_ref[0][:, 0:1] + deg_ref[1][:, 0:1], 1.0)
    dgi = jnp.maximum(deg_ref[0][:, 8:9] + deg_ref[1][:, 8:9], 1.0)
    a = lax.rsqrt(dgo)
    b = lax.rsqrt(dgi)
    c_ref[...] = a * b
    ia_ref[...] = jnp.sqrt(dgo)
    row = i * _RBP + lax.broadcasted_iota(jnp.int32, (_RBP, 1), 0)
    u0 = jnp.where(row < N_NODES, a * g0, 0.0)
    u0_ref[0] = u0[:, :HC]
    u0_ref[1] = u0[:, HC:]


_prep_call = pl.pallas_call(
    _prep_body,
    grid=(NPAD // _RBP,),
    in_specs=[
        pl.BlockSpec((_RBP, D_FEAT), lambda i: (i, 0)),
        pl.BlockSpec((D_FEAT, D_HID), lambda i: (0, 0)),
        pl.BlockSpec((1, D_HID), lambda i: (0, 0)),
        pl.BlockSpec((D_HID, N_CLS), lambda i: (0, 0)),
        pl.BlockSpec((NC, _RBP, 16), lambda i: (0, i, 0)),
    ],
    out_specs=(
        pl.BlockSpec((NC, _RBP, HC), lambda i: (0, i, 0)),
        pl.BlockSpec((_RBP, 1), lambda i: (i, 0)),
        pl.BlockSpec((_RBP, 1), lambda i: (i, 0)),
    ),
    out_shape=(
        jax.ShapeDtypeStruct((NC, NPAD, HC), jnp.float32),
        jax.ShapeDtypeStruct((NPAD, 1), jnp.float32),
        jax.ShapeDtypeStruct((NPAD, 1), jnp.float32),
    ),
)


def _fin_body(u8_ref, ia_ref, b1_ref, w2_ref, b2_ref, o_ref):
    g8 = ia_ref[...] * jnp.concatenate([u8_ref[0], u8_ref[1]], axis=1)
    z = jnp.maximum(g8 + b1_ref[...], 0.0)
    o_ref[...] = jnp.dot(z, w2_ref[...],
                         preferred_element_type=jnp.float32) + b2_ref[...]


_fin_call = pl.pallas_call(
    _fin_body,
    grid=(N_NODES // _RB,),
    in_specs=[
        pl.BlockSpec((NC, _RB, HC), lambda i: (0, i, 0)),
        pl.BlockSpec((_RB, 1), lambda i: (i, 0)),
        pl.BlockSpec((1, N_CLS), lambda i: (0, 0)),
        pl.BlockSpec((N_CLS, N_CLS), lambda i: (0, 0)),
        pl.BlockSpec((1, N_CLS), lambda i: (0, 0)),
    ],
    out_specs=pl.BlockSpec((_RB, N_CLS), lambda i: (i, 0)),
    out_shape=jax.ShapeDtypeStruct((N_NODES, N_CLS), jnp.float32),
)


def kernel(features, edge_index, W0, b0, W1, b1, W2, b2):
    src = jnp.asarray(edge_index[0], jnp.int32)
    dst = jnp.asarray(edge_index[1], jnp.int32)
    npd = TOT_E - N_EDGES
    pad = N_NODES + (jnp.arange(npd, dtype=jnp.int32) % NPAD_EXTRA)
    src2 = jnp.concatenate([src, pad]).reshape(TOT_E // CHUNK, CHUNK)
    dst2 = jnp.concatenate([dst, pad]).reshape(TOT_E // CHUNK, CHUNK)

    colid = jnp.arange(16, dtype=jnp.int32)
    ones2 = jnp.stack([jnp.tile((colid < 8).astype(jnp.float32), (CHUNK, 1)),
                       jnp.tile((colid >= 8).astype(jnp.float32), (CHUNK, 1))])
    zeros16 = jnp.zeros((NPAD, 16), jnp.float32)
    zerosHC = jnp.zeros((CHUNK, HC), jnp.float32)

    deg_p = _deg_call(src2, dst2, ones2, zeros16)
    u0p, cvec, inva = _prep_call(features, W0, b0.reshape(1, -1), W1, deg_p)
    cp = cvec.reshape(NPAD)

    pk2 = src2 * 16384 + dst2
    u8 = _prop_call(u0p, cp, pk2, zerosHC)
    return _fin_call(u8, inva, b1.reshape(1, -1), W2, b2.reshape(1, -1))


# speedup vs baseline: 25.2873x; 1.0263x over previous
import functools

import jax
import jax.numpy as jnp
from jax import lax
from jax.experimental import pallas as pl
from jax.experimental.pallas import tpu as pltpu
from jax.experimental.pallas import tpu_sc as plsc

N_NODES = 10000
N_EDGES = 320000
D_FEAT = 128
D_HID = 128
N_CLS = 64
K_LAYERS = 8
ALPHA = 1.0
LAMBD = 1.0
COEF = 1.0 / (1.0 + ALPHA * LAMBD)
K1 = COEF * ALPHA * LAMBD
K2 = COEF

NC = 2
NS = 16
NW = NC * NS
HC = N_CLS // NC
NPAD = 10240
NPAD_EXTRA = NPAD - N_NODES
SLICE = NPAD // NS
CHUNK = 128
TOT_E = 327680
EPT = TOT_E // NS
CPT = EPT // CHUNK
DEG_CPT = TOT_E // NW // CHUNK
NBUF = 4

_mesh = plsc.VectorSubcoreMesh(core_axis_name="c", subcore_axis_name="s",
                               num_cores=NC, num_subcores=NS)
_sc_params = pltpu.CompilerParams(use_tc_tiling_on_sc=False)


def _deg_body(src_hbm, dst_hbm, ones_hbm, zeros_hbm, deg_hbm,
              srcv, dstv, oneslo, oneshi, deg_sh, dsem):
    c = lax.axis_index("c")
    s = lax.axis_index("s")
    w = c * NS + s
    pltpu.sync_copy(src_hbm.at[pl.ds(w * DEG_CPT, DEG_CPT)], srcv)
    pltpu.sync_copy(dst_hbm.at[pl.ds(w * DEG_CPT, DEG_CPT)], dstv)
    pltpu.sync_copy(ones_hbm.at[0], oneslo)
    pltpu.sync_copy(ones_hbm.at[1], oneshi)
    pltpu.sync_copy(zeros_hbm.at[pl.ds(s * SLICE, SLICE)],
                    deg_sh.at[pl.ds(s * SLICE, SLICE)])
    plsc.subcore_barrier()

    def chunk(j, carry):
        pltpu.async_copy(oneslo, deg_sh.at[srcv.at[j]], dsem, add=True)
        pltpu.async_copy(oneshi, deg_sh.at[dstv.at[j]], dsem, add=True)
        return carry

    lax.fori_loop(0, DEG_CPT, chunk, 0)

    def drain(j, carry):
        pltpu.make_async_copy(ones_hbm.at[0], oneslo, dsem).wait()
        return carry

    lax.fori_loop(0, 2 * DEG_CPT, drain, 0)
    plsc.subcore_barrier()
    pltpu.sync_copy(deg_sh.at[pl.ds(s * SLICE, SLICE)],
                    deg_hbm.at[c, pl.ds(s * SLICE, SLICE)])


_deg_call = pl.kernel(
    _deg_body,
    out_type=jax.ShapeDtypeStruct((NC, NPAD, 16), jnp.float32),
    mesh=_mesh,
    scratch_types=[
        pltpu.VMEM((DEG_CPT, CHUNK), jnp.int32),
        pltpu.VMEM((DEG_CPT, CHUNK), jnp.int32),
        pltpu.VMEM((CHUNK, 16), jnp.float32),
        pltpu.VMEM((CHUNK, 16), jnp.float32),
        pltpu.VMEM_SHARED((NPAD, 16), jnp.float32),
        pltpu.SemaphoreType.DMA,
    ],
    compiler_params=_sc_params,
)


def _prop_body(u0_hbm, c_hbm, pk_hbm, zeros_hbm, u8_hbm,
               pkv, sring, dring, rowsv, u0v, cv, zerov, u_sh, agg_sh,
               gsem_a, gsem_b, ssem_a, ssem_b):
    c = lax.axis_index("c")
    s = lax.axis_index("s")
    nsl = pl.ds(s * SLICE, SLICE)
    pltpu.sync_copy(pk_hbm.at[pl.ds(s * CPT, CPT)], pkv)
    pltpu.sync_copy(u0_hbm.at[c, nsl], u0v)
    pltpu.sync_copy(u0_hbm.at[c, nsl], u_sh.at[nsl])
    pltpu.sync_copy(c_hbm.at[nsl], cv)
    pltpu.sync_copy(zeros_hbm, zerov)

    def zero_agg():
        for z in range(SLICE // CHUNK):
            pltpu.sync_copy(zerov,
                            agg_sh.at[pl.ds(s * SLICE + z * CHUNK, CHUNK)])

    zero_agg()
    plsc.subcore_barrier()

    def fire_g(g, bank, sem):
        for t in range(NBUF):
            j = g * NBUF + t
            for v in range(CHUNK // 16):
                slc = pl.ds(v * 16, 16)
                pk = pkv[j, slc]
                sring[bank, t, slc] = lax.shift_right_logical(pk, 14)
                dring[bank, t, slc] = lax.bitwise_and(pk, 16383)
            pltpu.async_copy(u_sh.at[sring.at[bank, t]],
                             rowsv.at[bank, t], sem)

    def fire_s(g, bank, sem):
        for t in range(NBUF):
            pltpu.async_copy(rowsv.at[bank, t],
                             agg_sh.at[dring.at[bank, t]], sem, add=True)

    def wait_n(sem):
        for _ in range(NBUF):
            pltpu.make_async_copy(u0_hbm.at[0, pl.ds(0, CHUNK)],
                                  rowsv.at[0, 0], sem).wait()

    ngroups = CPT // NBUF

    def one_iter(k, carry):
        fire_g(0, 0, gsem_a)
        fire_g(1, 1, gsem_b)
        wait_n(gsem_a)
        fire_s(0, 0, ssem_a)

        def body(ii, cc):
            g0 = ii * 2
            wait_n(ssem_a)
            fire_g(g0, 0, gsem_a)
            wait_n(gsem_b)
            fire_s(g0 - 1, 1, ssem_b)
            wait_n(ssem_b)
            fire_g(g0 + 1, 1, gsem_b)
            wait_n(gsem_a)
            fire_s(g0, 0, ssem_a)
            return cc

        lax.fori_loop(1, ngroups // 2, body, 0)
        wait_n(ssem_a)
        wait_n(gsem_b)
        fire_s(ngroups - 1, 1, ssem_b)
        wait_n(ssem_b)
        plsc.subcore_barrier()

        pieces = [(p // NBUF, p % NBUF) for p in range(SLICE // CHUNK)]
        for p, (pb, pt) in enumerate(pieces):
            pltpu.async_copy(agg_sh.at[pl.ds(s * SLICE + p * CHUNK, CHUNK)],
                             rowsv.at[pb, pt], gsem_a)
        for _ in pieces:
            pltpu.make_async_copy(u0_hbm.at[0, pl.ds(0, CHUNK)],
                                  rowsv.at[0, 0], gsem_a).wait()
        for p in range(len(pieces)):
            pltpu.async_copy(zerov,
                             agg_sh.at[pl.ds(s * SLICE + p * CHUNK, CHUNK)],
                             ssem_a)

        for p, (pb, pt) in enumerate(pieces):
            def upd(n16, cc, p=p, pb=pb, pt=pt):
                cvec = cv[pl.ds(p * CHUNK + n16 * 16, 16)] * K1
                for j in range(16):
                    n = n16 * 16 + j
                    cn = cvec[j]
                    for h in range(HC // 16):
                        slc = pl.ds(h * 16, 16)
                        rowsv[pb, pt, n, slc] = (cn * rowsv[pb, pt, n, slc]
                                                 + K2 * u0v[p * CHUNK + n, slc])
                return cc

            lax.fori_loop(0, CHUNK // 16, upd, 0)
            pltpu.async_copy(rowsv.at[pb, pt],
                             u_sh.at[pl.ds(s * SLICE + p * CHUNK, CHUNK)],
                             ssem_b)
        for _ in pieces:
            pltpu.make_async_copy(u0_hbm.at[0, pl.ds(0, CHUNK)],
                                  rowsv.at[0, 0], ssem_a).wait()
            pltpu.make_async_copy(u0_hbm.at[0, pl.ds(0, CHUNK)],
                                  rowsv.at[0, 0], ssem_b).wait()
        plsc.subcore_barrier()
        return carry

    lax.fori_loop(0, K_LAYERS, one_iter, 0)
    pltpu.sync_copy(u_sh.at[nsl], u8_hbm.at[c, nsl])


_prop_call = pl.kernel(
    _prop_body,
    out_type=jax.ShapeDtypeStruct((NC, NPAD, HC), jnp.float32),
    mesh=_mesh,
    scratch_types=[
        pltpu.VMEM((CPT, CHUNK), jnp.int32),
        pltpu.VMEM((2, NBUF, CHUNK), jnp.int32),
        pltpu.VMEM((2, NBUF, CHUNK), jnp.int32),
        pltpu.VMEM((2, NBUF, CHUNK, HC), jnp.float32),
        pltpu.VMEM((SLICE, HC), jnp.float32),
        pltpu.VMEM((SLICE,), jnp.float32),
        pltpu.VMEM((CHUNK, HC), jnp.float32),
        pltpu.VMEM_SHARED((NPAD, HC), jnp.float32),
        pltpu.VMEM_SHARED((NPAD, HC), jnp.float32),
        pltpu.SemaphoreType.DMA,
        pltpu.SemaphoreType.DMA,
        pltpu.SemaphoreType.DMA,
        pltpu.SemaphoreType.DMA,
    ],
    compiler_params=_sc_params,
)


_RB = 1000
_RBP = 1024


def _mm_body(x_ref, w0_ref, b0_ref, w1_ref, g0_ref):
    h0 = jnp.dot(x_ref[...], w0_ref[...],
                 preferred_element_type=jnp.float32) + b0_ref[...]
    g0_ref[...] = jnp.dot(h0, w1_ref[...], preferred_element_type=jnp.float32)


_mm_call = pl.pallas_call(
    _mm_body,
    grid=(NPAD // _RBP,),
    in_specs=[
        pl.BlockSpec((_RBP, D_FEAT), lambda i: (i, 0)),
        pl.BlockSpec((D_FEAT, D_HID), lambda i: (0, 0)),
        pl.BlockSpec((1, D_HID), lambda i: (0, 0)),
        pl.BlockSpec((D_HID, N_CLS), lambda i: (0, 0)),
    ],
    out_specs=pl.BlockSpec((_RBP, N_CLS), lambda i: (i, 0)),
    out_shape=jax.ShapeDtypeStruct((NPAD, N_CLS), jnp.float32),
)


def _prep_body(g0_ref, deg_ref, u0_ref, c_ref, ia_ref):
    i = pl.program_id(0)
    g0 = g0_ref[...]
    dgo = jnp.maximum(deg_ref[0][:, 0:1] + deg_ref[1][:, 0:1], 1.0)
    dgi = jnp.maximum(deg_ref[0][:, 8:9] + deg_ref[1][:, 8:9], 1.0)
    a = lax.rsqrt(dgo)
    b = lax.rsqrt(dgi)
    c_ref[...] = a * b
    ia_ref[...] = jnp.sqrt(dgo)
    row = i * _RBP + lax.broadcasted_iota(jnp.int32, (_RBP, 1), 0)
    u0 = jnp.where(row < N_NODES, a * g0, 0.0)
    u0_ref[0] = u0[:, :HC]
    u0_ref[1] = u0[:, HC:]


_prep_call = pl.pallas_call(
    _prep_body,
    grid=(NPAD // _RBP,),
    in_specs=[
        pl.BlockSpec((_RBP, N_CLS), lambda i: (i, 0)),
        pl.BlockSpec((NC, _RBP, 16), lambda i: (0, i, 0)),
    ],
    out_specs=(
        pl.BlockSpec((NC, _RBP, HC), lambda i: (0, i, 0)),
        pl.BlockSpec((_RBP, 1), lambda i: (i, 0)),
        pl.BlockSpec((_RBP, 1), lambda i: (i, 0)),
    ),
    out_shape=(
        jax.ShapeDtypeStruct((NC, NPAD, HC), jnp.float32),
        jax.ShapeDtypeStruct((NPAD, 1), jnp.float32),
        jax.ShapeDtypeStruct((NPAD, 1), jnp.float32),
    ),
)


def _fin_body(u8_ref, ia_ref, b1_ref, w2_ref, b2_ref, o_ref):
    g8 = ia_ref[...] * jnp.concatenate([u8_ref[0], u8_ref[1]], axis=1)
    z = jnp.maximum(g8 + b1_ref[...], 0.0)
    o_ref[...] = jnp.dot(z, w2_ref[...],
                         preferred_element_type=jnp.float32) + b2_ref[...]


_fin_call = pl.pallas_call(
    _fin_body,
    grid=(N_NODES // _RB,),
    in_specs=[
        pl.BlockSpec((NC, _RB, HC), lambda i: (0, i, 0)),
        pl.BlockSpec((_RB, 1), lambda i: (i, 0)),
        pl.BlockSpec((1, N_CLS), lambda i: (0, 0)),
        pl.BlockSpec((N_CLS, N_CLS), lambda i: (0, 0)),
        pl.BlockSpec((1, N_CLS), lambda i: (0, 0)),
    ],
    out_specs=pl.BlockSpec((_RB, N_CLS), lambda i: (i, 0)),
    out_shape=jax.ShapeDtypeStruct((N_NODES, N_CLS), jnp.float32),
)


def kernel(features, edge_index, W0, b0, W1, b1, W2, b2):
    src = jnp.asarray(edge_index[0], jnp.int32)
    dst = jnp.asarray(edge_index[1], jnp.int32)
    npd = TOT_E - N_EDGES
    pad = N_NODES + (jnp.arange(npd, dtype=jnp.int32) % NPAD_EXTRA)
    src2 = jnp.concatenate([src, pad]).reshape(TOT_E // CHUNK, CHUNK)
    dst2 = jnp.concatenate([dst, pad]).reshape(TOT_E // CHUNK, CHUNK)

    colid = jnp.arange(16, dtype=jnp.int32)
    ones2 = jnp.stack([jnp.tile((colid < 8).astype(jnp.float32), (CHUNK, 1)),
                       jnp.tile((colid >= 8).astype(jnp.float32), (CHUNK, 1))])
    zeros16 = jnp.zeros((NPAD, 16), jnp.float32)
    zerosHC = jnp.zeros((CHUNK, HC), jnp.float32)

    deg_p = _deg_call(src2, dst2, ones2, zeros16)
    g0 = _mm_call(features, W0, b0.reshape(1, -1), W1)
    u0p, cvec, inva = _prep_call(g0, deg_p)
    cp = cvec.reshape(NPAD)

    pk2 = src2 * 16384 + dst2
    u8 = _prop_call(u0p, cp, pk2, zerosHC)
    return _fin_call(u8, inva, b1.reshape(1, -1), W2, b2.reshape(1, -1))


# speedup vs baseline: 25.3834x; 1.0038x over previous
import functools

import jax
import jax.numpy as jnp
from jax import lax
from jax.experimental import pallas as pl
from jax.experimental.pallas import tpu as pltpu
from jax.experimental.pallas import tpu_sc as plsc

N_NODES = 10000
N_EDGES = 320000
D_FEAT = 128
D_HID = 128
N_CLS = 64
K_LAYERS = 8
ALPHA = 1.0
LAMBD = 1.0
COEF = 1.0 / (1.0 + ALPHA * LAMBD)
K1 = COEF * ALPHA * LAMBD
K2 = COEF

NC = 2
NS = 16
NW = NC * NS
HC = N_CLS // NC
NPAD = 10240
NPAD_EXTRA = NPAD - N_NODES
SLICE = NPAD // NS
CHUNK = 128
TOT_E = 327680
EPT = TOT_E // NS
CPT = EPT // CHUNK
DEG_CPT = TOT_E // NW // CHUNK
NBUF = 5

_mesh = plsc.VectorSubcoreMesh(core_axis_name="c", subcore_axis_name="s",
                               num_cores=NC, num_subcores=NS)
_sc_params = pltpu.CompilerParams(use_tc_tiling_on_sc=False)


def _deg_body(src_hbm, dst_hbm, ones_hbm, zeros_hbm, deg_hbm,
              srcv, dstv, oneslo, oneshi, deg_sh, dsem):
    c = lax.axis_index("c")
    s = lax.axis_index("s")
    w = c * NS + s
    pltpu.sync_copy(src_hbm.at[pl.ds(w * DEG_CPT, DEG_CPT)], srcv)
    pltpu.sync_copy(dst_hbm.at[pl.ds(w * DEG_CPT, DEG_CPT)], dstv)
    pltpu.sync_copy(ones_hbm.at[0], oneslo)
    pltpu.sync_copy(ones_hbm.at[1], oneshi)
    pltpu.sync_copy(zeros_hbm.at[pl.ds(s * SLICE, SLICE)],
                    deg_sh.at[pl.ds(s * SLICE, SLICE)])
    plsc.subcore_barrier()

    def chunk(j, carry):
        pltpu.async_copy(oneslo, deg_sh.at[srcv.at[j]], dsem, add=True)
        pltpu.async_copy(oneshi, deg_sh.at[dstv.at[j]], dsem, add=True)
        return carry

    lax.fori_loop(0, DEG_CPT, chunk, 0)

    def drain(j, carry):
        pltpu.make_async_copy(ones_hbm.at[0], oneslo, dsem).wait()
        return carry

    lax.fori_loop(0, 2 * DEG_CPT, drain, 0)
    plsc.subcore_barrier()
    pltpu.sync_copy(deg_sh.at[pl.ds(s * SLICE, SLICE)],
                    deg_hbm.at[c, pl.ds(s * SLICE, SLICE)])


_deg_call = pl.kernel(
    _deg_body,
    out_type=jax.ShapeDtypeStruct((NC, NPAD, 16), jnp.float32),
    mesh=_mesh,
    scratch_types=[
        pltpu.VMEM((DEG_CPT, CHUNK), jnp.int32),
        pltpu.VMEM((DEG_CPT, CHUNK), jnp.int32),
        pltpu.VMEM((CHUNK, 16), jnp.float32),
        pltpu.VMEM((CHUNK, 16), jnp.float32),
        pltpu.VMEM_SHARED((NPAD, 16), jnp.float32),
        pltpu.SemaphoreType.DMA,
    ],
    compiler_params=_sc_params,
)


def _prop_body(u0_hbm, c_hbm, pk_hbm, zeros_hbm, u8_hbm,
               pkv, sring, dring, rowsv, u0v, cv, zerov, u_sh, agg_sh,
               gsem_a, gsem_b, ssem_a, ssem_b):
    c = lax.axis_index("c")
    s = lax.axis_index("s")
    nsl = pl.ds(s * SLICE, SLICE)
    pltpu.sync_copy(pk_hbm.at[pl.ds(s * CPT, CPT)], pkv)
    pltpu.sync_copy(u0_hbm.at[c, nsl], u0v)
    pltpu.sync_copy(u0_hbm.at[c, nsl], u_sh.at[nsl])
    pltpu.sync_copy(c_hbm.at[nsl], cv)
    pltpu.sync_copy(zeros_hbm, zerov)

    def zero_agg():
        for z in range(SLICE // CHUNK):
            pltpu.sync_copy(zerov,
                            agg_sh.at[pl.ds(s * SLICE + z * CHUNK, CHUNK)])

    zero_agg()
    plsc.subcore_barrier()

    def fire_g(g, bank, sem):
        for t in range(NBUF):
            j = g * NBUF + t
            for v in range(CHUNK // 16):
                slc = pl.ds(v * 16, 16)
                pk = pkv[j, slc]
                sring[bank, t, slc] = lax.shift_right_logical(pk, 14)
                dring[bank, t, slc] = lax.bitwise_and(pk, 16383)
            pltpu.async_copy(u_sh.at[sring.at[bank, t]],
                             rowsv.at[bank, t], sem)

    def fire_s(g, bank, sem):
        for t in range(NBUF):
            pltpu.async_copy(rowsv.at[bank, t],
                             agg_sh.at[dring.at[bank, t]], sem, add=True)

    def wait_n(sem):
        for _ in range(NBUF):
            pltpu.make_async_copy(u0_hbm.at[0, pl.ds(0, CHUNK)],
                                  rowsv.at[0, 0], sem).wait()

    ngroups = CPT // NBUF

    def one_iter(k, carry):
        fire_g(0, 0, gsem_a)
        fire_g(1, 1, gsem_b)
        wait_n(gsem_a)
        fire_s(0, 0, ssem_a)

        def body(ii, cc):
            g0 = ii * 2
            wait_n(ssem_a)
            fire_g(g0, 0, gsem_a)
            wait_n(gsem_b)
            fire_s(g0 - 1, 1, ssem_b)
            wait_n(ssem_b)
            fire_g(g0 + 1, 1, gsem_b)
            wait_n(gsem_a)
            fire_s(g0, 0, ssem_a)
            return cc

        lax.fori_loop(1, ngroups // 2, body, 0)
        wait_n(ssem_a)
        wait_n(gsem_b)
        fire_s(ngroups - 1, 1, ssem_b)
        wait_n(ssem_b)
        plsc.subcore_barrier()

        pieces = [(p // NBUF, p % NBUF) for p in range(SLICE // CHUNK)]
        for p, (pb, pt) in enumerate(pieces):
            pltpu.async_copy(agg_sh.at[pl.ds(s * SLICE + p * CHUNK, CHUNK)],
                             rowsv.at[pb, pt], gsem_a)
        for _ in pieces:
            pltpu.make_async_copy(u0_hbm.at[0, pl.ds(0, CHUNK)],
                                  rowsv.at[0, 0], gsem_a).wait()
        for p in range(len(pieces)):
            pltpu.async_copy(zerov,
                             agg_sh.at[pl.ds(s * SLICE + p * CHUNK, CHUNK)],
                             ssem_a)

        for p, (pb, pt) in enumerate(pieces):
            def upd(n16, cc, p=p, pb=pb, pt=pt):
                cvec = cv[pl.ds(p * CHUNK + n16 * 16, 16)] * K1
                for j in range(16):
                    n = n16 * 16 + j
                    cn = cvec[j]
                    for h in range(HC // 16):
                        slc = pl.ds(h * 16, 16)
                        rowsv[pb, pt, n, slc] = (cn * rowsv[pb, pt, n, slc]
                                                 + K2 * u0v[p * CHUNK + n, slc])
                return cc

            lax.fori_loop(0, CHUNK // 16, upd, 0)
            pltpu.async_copy(rowsv.at[pb, pt],
                             u_sh.at[pl.ds(s * SLICE + p * CHUNK, CHUNK)],
                             ssem_b)
        for _ in pieces:
            pltpu.make_async_copy(u0_hbm.at[0, pl.ds(0, CHUNK)],
                                  rowsv.at[0, 0], ssem_a).wait()
            pltpu.make_async_copy(u0_hbm.at[0, pl.ds(0, CHUNK)],
                                  rowsv.at[0, 0], ssem_b).wait()
        plsc.subcore_barrier()
        return carry

    lax.fori_loop(0, K_LAYERS, one_iter, 0)
    pltpu.sync_copy(u_sh.at[nsl], u8_hbm.at[c, nsl])


_prop_call = pl.kernel(
    _prop_body,
    out_type=jax.ShapeDtypeStruct((NC, NPAD, HC), jnp.float32),
    mesh=_mesh,
    scratch_types=[
        pltpu.VMEM((CPT, CHUNK), jnp.int32),
        pltpu.VMEM((2, NBUF, CHUNK), jnp.int32),
        pltpu.VMEM((2, NBUF, CHUNK), jnp.int32),
        pltpu.VMEM((2, NBUF, CHUNK, HC), jnp.float32),
        pltpu.VMEM((SLICE, HC), jnp.float32),
        pltpu.VMEM((SLICE,), jnp.float32),
        pltpu.VMEM((CHUNK, HC), jnp.float32),
        pltpu.VMEM_SHARED((NPAD, HC), jnp.float32),
        pltpu.VMEM_SHARED((NPAD, HC), jnp.float32),
        pltpu.SemaphoreType.DMA,
        pltpu.SemaphoreType.DMA,
        pltpu.SemaphoreType.DMA,
        pltpu.SemaphoreType.DMA,
    ],
    compiler_params=_sc_params,
)


_RB = 1000
_RBP = 1024


def _mm_body(x_ref, w0_ref, b0_ref, w1_ref, g0_ref):
    h0 = jnp.dot(x_ref[...], w0_ref[...],
                 preferred_element_type=jnp.float32) + b0_ref[...]
    g0_ref[...] = jnp.dot(h0, w1_ref[...], preferred_element_type=jnp.float32)


_mm_call = pl.pallas_call(
    _mm_body,
    grid=(NPAD // _RBP,),
    in_specs=[
        pl.BlockSpec((_RBP, D_FEAT), lambda i: (i, 0)),
        pl.BlockSpec((D_FEAT, D_HID), lambda i: (0, 0)),
        pl.BlockSpec((1, D_HID), lambda i: (0, 0)),
        pl.BlockSpec((D_HID, N_CLS), lambda i: (0, 0)),
    ],
    out_specs=pl.BlockSpec((_RBP, N_CLS), lambda i: (i, 0)),
    out_shape=jax.ShapeDtypeStruct((NPAD, N_CLS), jnp.float32),
)


def _prep_body(g0_ref, deg_ref, u0_ref, c_ref, ia_ref):
    i = pl.program_id(0)
    g0 = g0_ref[...]
    dgo = jnp.maximum(deg_ref[0][:, 0:1] + deg_ref[1][:, 0:1], 1.0)
    dgi = jnp.maximum(deg_ref[0][:, 8:9] + deg_ref[1][:, 8:9], 1.0)
    a = lax.rsqrt(dgo)
    b = lax.rsqrt(dgi)
    c_ref[...] = a * b
    ia_ref[...] = jnp.sqrt(dgo)
    row = i * _RBP + lax.broadcasted_iota(jnp.int32, (_RBP, 1), 0)
    u0 = jnp.where(row < N_NODES, a * g0, 0.0)
    u0_ref[0] = u0[:, :HC]
    u0_ref[1] = u0[:, HC:]


_prep_call = pl.pallas_call(
    _prep_body,
    grid=(NPAD // _RBP,),
    in_specs=[
        pl.BlockSpec((_RBP, N_CLS), lambda i: (i, 0)),
        pl.BlockSpec((NC, _RBP, 16), lambda i: (0, i, 0)),
    ],
    out_specs=(
        pl.BlockSpec((NC, _RBP, HC), lambda i: (0, i, 0)),
        pl.BlockSpec((_RBP, 1), lambda i: (i, 0)),
        pl.BlockSpec((_RBP, 1), lambda i: (i, 0)),
    ),
    out_shape=(
        jax.ShapeDtypeStruct((NC, NPAD, HC), jnp.float32),
        jax.ShapeDtypeStruct((NPAD, 1), jnp.float32),
        jax.ShapeDtypeStruct((NPAD, 1), jnp.float32),
    ),
)


def _fin_body(u8_ref, ia_ref, b1_ref, w2_ref, b2_ref, o_ref):
    g8 = ia_ref[...] * jnp.concatenate([u8_ref[0], u8_ref[1]], axis=1)
    z = jnp.maximum(g8 + b1_ref[...], 0.0)
    o_ref[...] = jnp.dot(z, w2_ref[...],
                         preferred_element_type=jnp.float32) + b2_ref[...]


_fin_call = pl.pallas_call(
    _fin_body,
    grid=(N_NODES // _RB,),
    in_specs=[
        pl.BlockSpec((NC, _RB, HC), lambda i: (0, i, 0)),
        pl.BlockSpec((_RB, 1), lambda i: (i, 0)),
        pl.BlockSpec((1, N_CLS), lambda i: (0, 0)),
        pl.BlockSpec((N_CLS, N_CLS), lambda i: (0, 0)),
        pl.BlockSpec((1, N_CLS), lambda i: (0, 0)),
    ],
    out_specs=pl.BlockSpec((_RB, N_CLS), lambda i: (i, 0)),
    out_shape=jax.ShapeDtypeStruct((N_NODES, N_CLS), jnp.float32),
)


def kernel(features, edge_index, W0, b0, W1, b1, W2, b2):
    src = jnp.asarray(edge_index[0], jnp.int32)
    dst = jnp.asarray(edge_index[1], jnp.int32)
    npd = TOT_E - N_EDGES
    pad = N_NODES + (jnp.arange(npd, dtype=jnp.int32) % NPAD_EXTRA)
    src2 = jnp.concatenate([src, pad]).reshape(TOT_E // CHUNK, CHUNK)
    dst2 = jnp.concatenate([dst, pad]).reshape(TOT_E // CHUNK, CHUNK)

    colid = jnp.arange(16, dtype=jnp.int32)
    ones2 = jnp.stack([jnp.tile((colid < 8).astype(jnp.float32), (CHUNK, 1)),
                       jnp.tile((colid >= 8).astype(jnp.float32), (CHUNK, 1))])
    zeros16 = jnp.zeros((NPAD, 16), jnp.float32)
    zerosHC = jnp.zeros((CHUNK, HC), jnp.float32)

    deg_p = _deg_call(src2, dst2, ones2, zeros16)
    g0 = _mm_call(features, W0, b0.reshape(1, -1), W1)
    u0p, cvec, inva = _prep_call(g0, deg_p)
    cp = cvec.reshape(NPAD)

    pk2 = src2 * 16384 + dst2
    u8 = _prop_call(u0p, cp, pk2, zerosHC)
    return _fin_call(u8, inva, b1.reshape(1, -1), W2, b2.reshape(1, -1))
